# baseline (speedup 1.0000x reference)
import functools

import jax
import jax.numpy as jnp
from jax import lax
from jax.experimental import pallas as pl
from jax.experimental.pallas import tpu as pltpu
from jax.experimental.pallas import tpu_sc as plsc

DIM = 32
LANE = 16
BLK = 128


@functools.lru_cache(maxsize=None)
def _make_detile(vocab: int):
    info = plsc.get_sparse_core_info()
    nc, ns = info.num_cores, info.num_subcores
    nw = nc * ns
    nblk = vocab // BLK
    tail = vocab - nblk * BLK
    per_w = nblk // nw
    nrem = nblk - per_w * nw
    assert per_w % 2 == 0 and tail % 4 == 0 and vocab % 8 == 0

    mesh = plsc.VectorSubcoreMesh(core_axis_name="c", subcore_axis_name="s")

    @functools.partial(
        pl.kernel,
        mesh=mesh,
        compiler_params=pltpu.CompilerParams(use_tc_tiling_on_sc=True, needs_layout_passes=False),
        out_type=jax.ShapeDtypeStruct((vocab // 4, BLK), jnp.float32),
        scratch_types=[
            pltpu.VMEM((2, DIM, BLK), jnp.float32),
            pltpu.VMEM((2, DIM, BLK), jnp.float32),
            pltpu.VMEM((DIM, tail), jnp.float32),
            pltpu.SemaphoreType.DMA,
            pltpu.SemaphoreType.DMA,
            pltpu.SemaphoreType.DMA,
            pltpu.SemaphoreType.DMA,
        ],
    )
    def detile_kernel(tab_t, tab_l, tin, tasm, tin_part,
                      lsem0, lsem1, wsem0, wsem1):
        lsems = (lsem0, lsem1)
        wsems = (wsem0, wsem1)
        wid = lax.axis_index("s") * nc + lax.axis_index("c")
        w0 = wid * per_w
        iota = lax.broadcasted_iota(jnp.int32, (LANE,), 0)

        def fire_load(v, b):
            off = pl.multiple_of(v * BLK, BLK)
            pltpu.async_copy(tab_t.at[:, pl.ds(off, BLK)], tin.at[b], lsems[b])

        def wait_load(b):
            pltpu.make_async_copy(
                tab_t.at[:, pl.ds(0, BLK)], tin.at[b], lsems[b]).wait()

        def assemble(b):
            for r in range(BLK):
                rsp = jnp.full((LANE,), r, jnp.int32)
                for h in (0, LANE):
                    val = plsc.load_gather(tin.at[b], [iota + h, rsp])
                    tasm[b, r // 4, pl.ds((r % 4) * DIM + h, LANE)] = val

        def fire_write(v, b):
            off = pl.multiple_of(v * DIM, 8)
            pltpu.async_copy(tasm.at[b], tab_l.at[pl.ds(off, DIM)], wsems[b])

        def wait_write(b):
            pltpu.make_async_copy(
                tasm.at[b], tab_l.at[pl.ds(0, DIM)], wsems[b]).wait()

        fire_load(w0, 0)

        def body(u, c):
            for par in (0, 1):
                v = 2 * u + par
                nb = 1 - par

                @pl.when(v + 1 < per_w)
                def _():
                    fire_load(w0 + v + 1, nb)

                wait_load(par)

                @pl.when(v >= 2)
                def _():
                    wait_write(par)

                assemble(par)
                fire_write(w0 + v, par)
            return c

        lax.fori_loop(0, per_w // 2, body, 0)
        wait_write(0)
        wait_write(1)

        @pl.when(wid < nrem)
        def _():
            v = per_w * nw + wid
            fire_load(v, 0)
            wait_load(0)
            assemble(0)
            fire_write(v, 0)
            wait_write(0)

        @pl.when(wid == nrem)
        def _():
            pltpu.sync_copy(tab_t.at[:, pl.ds(nblk * BLK, tail)], tin_part)
            for r in range(tail):
                rsp = jnp.full((LANE,), r, jnp.int32)
                for h in (0, LANE):
                    val = plsc.load_gather(tin_part, [iota + h, rsp])
                    tasm[0, r // 4, pl.ds((r % 4) * DIM + h, LANE)] = val
            pltpu.sync_copy(tasm.at[0].at[pl.ds(0, tail // 4)],
                            tab_l.at[pl.ds(nblk * DIM, tail // 4)])

    return detile_kernel


@functools.lru_cache(maxsize=None)
def _make_gather(batch: int, hist: int, vocab: int):
    info = plsc.get_sparse_core_info()
    nc, ns = info.num_cores, info.num_subcores
    nw = nc * ns
    nblk = batch // BLK
    blk_per_w = nblk // nw
    assert blk_per_w * nw == nblk and hist % 2 == 0

    mesh = plsc.VectorSubcoreMesh(core_axis_name="c", subcore_axis_name="s")

    @functools.partial(
        pl.kernel,
        mesh=mesh,
        compiler_params=pltpu.CompilerParams(use_tc_tiling_on_sc=True, needs_layout_passes=False),
        out_type=jax.ShapeDtypeStruct((hist, DIM, batch), jnp.float32),
        scratch_types=[
            pltpu.VMEM((hist, BLK), jnp.int32),
            pltpu.VMEM((2, BLK), jnp.int32),
            pltpu.VMEM((2, BLK), jnp.int32),
            pltpu.VMEM((2, BLK, BLK), jnp.float32),
            pltpu.VMEM((2, DIM, BLK), jnp.float32),
            pltpu.SemaphoreType.DMA,
            pltpu.SemaphoreType.DMA,
            pltpu.SemaphoreType.DMA,
            pltpu.SemaphoreType.DMA,
        ],
    )
    def gather_kernel(idx_t, tab_l, out_t, idxb, idx2, rem32, rows, oasm,
                      gsem0, gsem1, wsem0, wsem1):
        gsems = (gsem0, gsem1)
        wsems = (wsem0, wsem1)
        wid = lax.axis_index("s") * nc + lax.axis_index("c")
        iota = lax.broadcasted_iota(jnp.int32, (LANE,), 0)

        def blk_body(blk, carry):
            b0 = pl.multiple_of((wid * blk_per_w + blk) * BLK, BLK)
            pltpu.sync_copy(idx_t.at[:, pl.ds(b0, BLK)], idxb)

            def compute_idx(t, b):
                for g in range(8):
                    v = idxb[t, pl.ds(g * LANE, LANE)]
                    idx2[b, pl.ds(g * LANE, LANE)] = v >> 2
                    rem32[b, pl.ds(g * LANE, LANE)] = (v & 3) * DIM

            def fire_gather(b):
                pltpu.async_copy(tab_l.at[idx2.at[b]], rows.at[b], gsems[b])

            def wait_gather(b):
                pltpu.make_async_copy(
                    tab_l.at[pl.ds(0, BLK)], rows.at[b], gsems[b]).wait()

            def assemble(b):
                for g in range(8):
                    rowi = iota + g * LANE
                    rem_g = rem32[b, pl.ds(g * LANE, LANE)]
                    for d in range(DIM):
                        val = plsc.load_gather(rows.at[b], [rowi, rem_g + d])
                        oasm[b, d, pl.ds(g * LANE, LANE)] = val

            def fire_write(t, b):
                pltpu.async_copy(
                    oasm.at[b], out_t.at[t, :, pl.ds(b0, BLK)], wsems[b])

            def wait_write(b):
                pltpu.make_async_copy(
                    oasm.at[b], out_t.at[0, :, pl.ds(0, BLK)], wsems[b]).wait()

            compute_idx(0, 0)
            fire_gather(0)

            def body(u, c):
                for par in (0, 1):
                    t = 2 * u + par
                    nb = 1 - par

                    @pl.when(t + 1 < hist)
                    def _():
                        compute_idx(t + 1, nb)
                        fire_gather(nb)

                    wait_gather(par)

                    @pl.when(t >= 2)
                    def _():
                        wait_write(par)

                    assemble(par)
                    fire_write(t, par)
                return c

            lax.fori_loop(0, hist // 2, body, 0)
            wait_write(0)
            wait_write(1)
            return carry

        lax.fori_loop(0, blk_per_w, blk_body, 0)

    return gather_kernel


def kernel(inputs, table):
    batch, hist = inputs.shape
    vocab, dim = table.shape
    idx_t = jnp.swapaxes(jnp.asarray(inputs, jnp.int32), 0, 1)
    tab_t = jnp.swapaxes(table, 0, 1)
    tab_l = _make_detile(vocab)(tab_t)
    out_t = _make_gather(batch, hist, vocab)(idx_t, tab_l)
    return jnp.transpose(out_t, (2, 0, 1))


# speedup vs baseline: 1.6780x; 1.6780x over previous
import functools

import jax
import jax.numpy as jnp
from jax import lax
from jax.experimental import pallas as pl
from jax.experimental.pallas import tpu as pltpu
from jax.experimental.pallas import tpu_sc as plsc

DIM = 32
LANE = 16
BLK = 128


@functools.lru_cache(maxsize=None)
def _make_detile(vocab: int):
    info = plsc.get_sparse_core_info()
    nc, ns = info.num_cores, info.num_subcores
    nw = nc * ns
    nblk = vocab // BLK
    tail = vocab - nblk * BLK
    per_w = nblk // nw
    nrem = nblk - per_w * nw
    assert per_w % 2 == 0 and tail % 4 == 0 and vocab % 8 == 0

    mesh = plsc.VectorSubcoreMesh(core_axis_name="c", subcore_axis_name="s")

    @functools.partial(
        pl.kernel,
        mesh=mesh,
        compiler_params=pltpu.CompilerParams(use_tc_tiling_on_sc=True, needs_layout_passes=False),
        out_type=jax.ShapeDtypeStruct((vocab // 4, BLK), jnp.float32),
        scratch_types=[
            pltpu.VMEM((2, DIM, BLK), jnp.float32),
            pltpu.VMEM((2, DIM, BLK), jnp.float32),
            pltpu.VMEM((DIM, tail), jnp.float32),
            pltpu.SemaphoreType.DMA,
            pltpu.SemaphoreType.DMA,
            pltpu.SemaphoreType.DMA,
            pltpu.SemaphoreType.DMA,
        ],
    )
    def detile_kernel(tab_t, tab_l, tin, tasm, tin_part,
                      lsem0, lsem1, wsem0, wsem1):
        lsems = (lsem0, lsem1)
        wsems = (wsem0, wsem1)
        wid = lax.axis_index("s") * nc + lax.axis_index("c")
        w0 = wid * per_w
        iota = lax.broadcasted_iota(jnp.int32, (LANE,), 0)

        def fire_load(v, b):
            off = pl.multiple_of(v * BLK, BLK)
            pltpu.async_copy(tab_t.at[:, pl.ds(off, BLK)], tin.at[b], lsems[b])

        def wait_load(b):
            pltpu.make_async_copy(
                tab_t.at[:, pl.ds(0, BLK)], tin.at[b], lsems[b]).wait()

        def assemble(b):
            ops = [(r, h) for r in range(BLK) for h in (0, LANE)]
            for c0 in range(0, len(ops), 16):
                chunk = ops[c0:c0 + 16]
                vals = [
                    plsc.load_gather(
                        tin.at[b], [iota + h, jnp.full((LANE,), r, jnp.int32)])
                    for (r, h) in chunk
                ]
                for (r, h), val in zip(chunk, vals):
                    tasm[b, r // 4, pl.ds((r % 4) * DIM + h, LANE)] = val

        def fire_write(v, b):
            off = pl.multiple_of(v * DIM, 8)
            pltpu.async_copy(tasm.at[b], tab_l.at[pl.ds(off, DIM)], wsems[b])

        def wait_write(b):
            pltpu.make_async_copy(
                tasm.at[b], tab_l.at[pl.ds(0, DIM)], wsems[b]).wait()

        fire_load(w0, 0)

        def body(u, c):
            for par in (0, 1):
                v = 2 * u + par
                nb = 1 - par

                @pl.when(v + 1 < per_w)
                def _():
                    fire_load(w0 + v + 1, nb)

                wait_load(par)

                @pl.when(v >= 2)
                def _():
                    wait_write(par)

                assemble(par)
                fire_write(w0 + v, par)
            return c

        lax.fori_loop(0, per_w // 2, body, 0)
        wait_write(0)
        wait_write(1)

        @pl.when(wid < nrem)
        def _():
            v = per_w * nw + wid
            fire_load(v, 0)
            wait_load(0)
            assemble(0)
            fire_write(v, 0)
            wait_write(0)

        @pl.when(wid == nrem)
        def _():
            pltpu.sync_copy(tab_t.at[:, pl.ds(nblk * BLK, tail)], tin_part)
            ops = [(r, h) for r in range(tail) for h in (0, LANE)]
            for c0 in range(0, len(ops), 16):
                chunk = ops[c0:c0 + 16]
                vals = [
                    plsc.load_gather(
                        tin_part, [iota + h, jnp.full((LANE,), r, jnp.int32)])
                    for (r, h) in chunk
                ]
                for (r, h), val in zip(chunk, vals):
                    tasm[0, r // 4, pl.ds((r % 4) * DIM + h, LANE)] = val
            pltpu.sync_copy(tasm.at[0].at[pl.ds(0, tail // 4)],
                            tab_l.at[pl.ds(nblk * DIM, tail // 4)])

    return detile_kernel


@functools.lru_cache(maxsize=None)
def _make_gather(batch: int, hist: int, vocab: int):
    info = plsc.get_sparse_core_info()
    nc, ns = info.num_cores, info.num_subcores
    nw = nc * ns
    nblk = batch // BLK
    blk_per_w = nblk // nw
    assert blk_per_w * nw == nblk and hist % 2 == 0

    mesh = plsc.VectorSubcoreMesh(core_axis_name="c", subcore_axis_name="s")

    @functools.partial(
        pl.kernel,
        mesh=mesh,
        compiler_params=pltpu.CompilerParams(use_tc_tiling_on_sc=True, needs_layout_passes=False),
        out_type=jax.ShapeDtypeStruct((hist, DIM, batch), jnp.float32),
        scratch_types=[
            pltpu.VMEM((hist, BLK), jnp.int32),
            pltpu.VMEM((2, BLK), jnp.int32),
            pltpu.VMEM((2, BLK), jnp.int32),
            pltpu.VMEM((2, BLK, BLK), jnp.float32),
            pltpu.VMEM((2, DIM, BLK), jnp.float32),
            pltpu.SemaphoreType.DMA,
            pltpu.SemaphoreType.DMA,
            pltpu.SemaphoreType.DMA,
            pltpu.SemaphoreType.DMA,
        ],
    )
    def gather_kernel(idx_t, tab_l, out_t, idxb, idx2, rem32, rows, oasm,
                      gsem0, gsem1, wsem0, wsem1):
        gsems = (gsem0, gsem1)
        wsems = (wsem0, wsem1)
        wid = lax.axis_index("s") * nc + lax.axis_index("c")
        iota = lax.broadcasted_iota(jnp.int32, (LANE,), 0)

        def blk_body(blk, carry):
            b0 = pl.multiple_of((wid * blk_per_w + blk) * BLK, BLK)
            pltpu.sync_copy(idx_t.at[:, pl.ds(b0, BLK)], idxb)

            def compute_idx(t, b):
                for g in range(8):
                    v = idxb[t, pl.ds(g * LANE, LANE)]
                    idx2[b, pl.ds(g * LANE, LANE)] = v >> 2
                    rem32[b, pl.ds(g * LANE, LANE)] = (v & 3) * DIM

            def fire_gather(b):
                pltpu.async_copy(tab_l.at[idx2.at[b]], rows.at[b], gsems[b])

            def wait_gather(b):
                pltpu.make_async_copy(
                    tab_l.at[pl.ds(0, BLK)], rows.at[b], gsems[b]).wait()

            def assemble(b):
                for g in range(8):
                    rowi = iota + g * LANE
                    rem_g = rem32[b, pl.ds(g * LANE, LANE)]
                    for d0 in (0, LANE):
                        vals = [
                            plsc.load_gather(rows.at[b], [rowi, rem_g + (d0 + k)])
                            for k in range(LANE)
                        ]
                        for k, val in enumerate(vals):
                            oasm[b, d0 + k, pl.ds(g * LANE, LANE)] = val

            def fire_write(t, b):
                pltpu.async_copy(
                    oasm.at[b], out_t.at[t, :, pl.ds(b0, BLK)], wsems[b])

            def wait_write(b):
                pltpu.make_async_copy(
                    oasm.at[b], out_t.at[0, :, pl.ds(0, BLK)], wsems[b]).wait()

            compute_idx(0, 0)
            fire_gather(0)

            def body(u, c):
                for par in (0, 1):
                    t = 2 * u + par
                    nb = 1 - par

                    @pl.when(t + 1 < hist)
                    def _():
                        compute_idx(t + 1, nb)
                        fire_gather(nb)

                    wait_gather(par)

                    @pl.when(t >= 2)
                    def _():
                        wait_write(par)

                    assemble(par)
                    fire_write(t, par)
                return c

            lax.fori_loop(0, hist // 2, body, 0)
            wait_write(0)
            wait_write(1)
            return carry

        lax.fori_loop(0, blk_per_w, blk_body, 0)

    return gather_kernel


def kernel(inputs, table):
    batch, hist = inputs.shape
    vocab, dim = table.shape
    idx_t = jnp.swapaxes(jnp.asarray(inputs, jnp.int32), 0, 1)
    tab_t = jnp.swapaxes(table, 0, 1)
    tab_l = _make_detile(vocab)(tab_t)
    out_t = _make_gather(batch, hist, vocab)(idx_t, tab_l)
    return jnp.transpose(out_t, (2, 0, 1))
